# Initial kernel scaffold; baseline (speedup 1.0000x reference)
import jax
import jax.numpy as jnp
from jax.experimental import pallas as pl


def kernel(in_feature, adj, W0, b0, W1, b1):
    raise NotImplementedError("write your pallas kernel here")


# speedup vs baseline: 1.0055x; 1.0055x over previous
import functools

import jax
import jax.numpy as jnp
from jax.experimental import pallas as pl
from jax.experimental.pallas import tpu as pltpu


def _xw_body(x_ref, w_ref, o_ref):
    o_ref[...] = jnp.dot(x_ref[...], w_ref[...],
                         preferred_element_type=jnp.float32)


def _spmm_body(adj_ref, s_ref, b_ref, h_ref, cs_ref):
    h = jnp.dot(adj_ref[...], s_ref[...], preferred_element_type=jnp.float32)
    h = jnp.maximum(h + b_ref[...], 0.0)
    h_ref[...] = h

    @pl.when(pl.program_id(0) == 0)
    def _init():
        cs_ref[...] = jnp.zeros_like(cs_ref)

    cs_ref[...] += jnp.sum(h, axis=0, keepdims=True)


def _norm_mm_body(h_ref, cs_ref, w_ref, o_ref, *, n_rows):
    x = h_ref[...] - cs_ref[...] * (1.0 / n_rows)
    rn = jnp.sqrt(1e-6 + jnp.sum(x * x, axis=1, keepdims=True))
    x = x / rn
    o_ref[...] = jnp.dot(x, w_ref[...], preferred_element_type=jnp.float32)


def _norm_body(h_ref, cs_ref, o_ref, *, n_rows):
    x = h_ref[...] - cs_ref[...] * (1.0 / n_rows)
    rn = jnp.sqrt(1e-6 + jnp.sum(x * x, axis=1, keepdims=True))
    o_ref[...] = x / rn


def _pick_block(n, target):
    best = 8
    for bm in range(8, min(n, target) + 1, 8):
        if n % bm == 0:
            best = bm
    return best


def kernel(in_feature, adj, W0, b0, W1, b1):
    n, d = in_feature.shape
    bm_big = _pick_block(n, 400)
    bm_small = _pick_block(n, 2000)

    b0r = b0.reshape(1, d)
    b1r = b1.reshape(1, d)

    xw = pl.pallas_call(
        _xw_body,
        grid=(n // bm_small,),
        in_specs=[
            pl.BlockSpec((bm_small, d), lambda i: (i, 0)),
            pl.BlockSpec((d, d), lambda i: (0, 0)),
        ],
        out_specs=pl.BlockSpec((bm_small, d), lambda i: (i, 0)),
        out_shape=jax.ShapeDtypeStruct((n, d), jnp.float32),
    )

    spmm = pl.pallas_call(
        _spmm_body,
        grid=(n // bm_big,),
        in_specs=[
            pl.BlockSpec((bm_big, n), lambda i: (i, 0)),
            pl.BlockSpec((n, d), lambda i: (0, 0)),
            pl.BlockSpec((1, d), lambda i: (0, 0)),
        ],
        out_specs=[
            pl.BlockSpec((bm_big, d), lambda i: (i, 0)),
            pl.BlockSpec((1, d), lambda i: (0, 0)),
        ],
        out_shape=[
            jax.ShapeDtypeStruct((n, d), jnp.float32),
            jax.ShapeDtypeStruct((1, d), jnp.float32),
        ],
    )

    norm_mm = pl.pallas_call(
        functools.partial(_norm_mm_body, n_rows=n),
        grid=(n // bm_small,),
        in_specs=[
            pl.BlockSpec((bm_small, d), lambda i: (i, 0)),
            pl.BlockSpec((1, d), lambda i: (0, 0)),
            pl.BlockSpec((d, d), lambda i: (0, 0)),
        ],
        out_specs=pl.BlockSpec((bm_small, d), lambda i: (i, 0)),
        out_shape=jax.ShapeDtypeStruct((n, d), jnp.float32),
    )

    norm = pl.pallas_call(
        functools.partial(_norm_body, n_rows=n),
        grid=(n // bm_small,),
        in_specs=[
            pl.BlockSpec((bm_small, d), lambda i: (i, 0)),
            pl.BlockSpec((1, d), lambda i: (0, 0)),
        ],
        out_specs=pl.BlockSpec((bm_small, d), lambda i: (i, 0)),
        out_shape=jax.ShapeDtypeStruct((n, d), jnp.float32),
    )

    s1 = xw(in_feature, W0)
    h1, cs1 = spmm(adj, s1, b0r)
    s2 = norm_mm(h1, cs1, W1)
    h2, cs2 = spmm(adj, s2, b1r)
    return norm(h2, cs2)


# speedup vs baseline: 1.0080x; 1.0025x over previous
import functools

import jax
import jax.numpy as jnp
from jax.experimental import pallas as pl
from jax.experimental.pallas import tpu as pltpu


def _xw_body(x_ref, w_ref, o_ref):
    o_ref[...] = jnp.dot(x_ref[...], w_ref[...],
                         preferred_element_type=jnp.float32)


def _spmm_body(adj_ref, s_ref, b_ref, h_ref, cs_ref):
    h = jnp.dot(adj_ref[...].astype(jnp.bfloat16),
                s_ref[...].astype(jnp.bfloat16),
                preferred_element_type=jnp.float32)
    h = jnp.maximum(h + b_ref[...], 0.0)
    h_ref[...] = h

    @pl.when(pl.program_id(0) == 0)
    def _init():
        cs_ref[...] = jnp.zeros_like(cs_ref)

    cs_ref[...] += jnp.sum(h, axis=0, keepdims=True)


def _norm_mm_body(h_ref, cs_ref, w_ref, o_ref, *, n_rows):
    x = h_ref[...] - cs_ref[...] * (1.0 / n_rows)
    rn = jnp.sqrt(1e-6 + jnp.sum(x * x, axis=1, keepdims=True))
    x = x / rn
    o_ref[...] = jnp.dot(x, w_ref[...], preferred_element_type=jnp.float32)


def _norm_body(h_ref, cs_ref, o_ref, *, n_rows):
    x = h_ref[...] - cs_ref[...] * (1.0 / n_rows)
    rn = jnp.sqrt(1e-6 + jnp.sum(x * x, axis=1, keepdims=True))
    o_ref[...] = x / rn


def _pick_block(n, target):
    best = 8
    for bm in range(8, min(n, target) + 1, 8):
        if n % bm == 0:
            best = bm
    return best


def kernel(in_feature, adj, W0, b0, W1, b1):
    n, d = in_feature.shape
    bm_big = _pick_block(n, 400)
    bm_small = _pick_block(n, 2000)

    b0r = b0.reshape(1, d)
    b1r = b1.reshape(1, d)

    xw = pl.pallas_call(
        _xw_body,
        grid=(n // bm_small,),
        in_specs=[
            pl.BlockSpec((bm_small, d), lambda i: (i, 0)),
            pl.BlockSpec((d, d), lambda i: (0, 0)),
        ],
        out_specs=pl.BlockSpec((bm_small, d), lambda i: (i, 0)),
        out_shape=jax.ShapeDtypeStruct((n, d), jnp.float32),
    )

    spmm = pl.pallas_call(
        _spmm_body,
        grid=(n // bm_big,),
        in_specs=[
            pl.BlockSpec((bm_big, n), lambda i: (i, 0)),
            pl.BlockSpec((n, d), lambda i: (0, 0)),
            pl.BlockSpec((1, d), lambda i: (0, 0)),
        ],
        out_specs=[
            pl.BlockSpec((bm_big, d), lambda i: (i, 0)),
            pl.BlockSpec((1, d), lambda i: (0, 0)),
        ],
        out_shape=[
            jax.ShapeDtypeStruct((n, d), jnp.float32),
            jax.ShapeDtypeStruct((1, d), jnp.float32),
        ],
    )

    norm_mm = pl.pallas_call(
        functools.partial(_norm_mm_body, n_rows=n),
        grid=(n // bm_small,),
        in_specs=[
            pl.BlockSpec((bm_small, d), lambda i: (i, 0)),
            pl.BlockSpec((1, d), lambda i: (0, 0)),
            pl.BlockSpec((d, d), lambda i: (0, 0)),
        ],
        out_specs=pl.BlockSpec((bm_small, d), lambda i: (i, 0)),
        out_shape=jax.ShapeDtypeStruct((n, d), jnp.float32),
    )

    norm = pl.pallas_call(
        functools.partial(_norm_body, n_rows=n),
        grid=(n // bm_small,),
        in_specs=[
            pl.BlockSpec((bm_small, d), lambda i: (i, 0)),
            pl.BlockSpec((1, d), lambda i: (0, 0)),
        ],
        out_specs=pl.BlockSpec((bm_small, d), lambda i: (i, 0)),
        out_shape=jax.ShapeDtypeStruct((n, d), jnp.float32),
    )

    s1 = xw(in_feature, W0)
    h1, cs1 = spmm(adj, s1, b0r)
    s2 = norm_mm(h1, cs1, W1)
    h2, cs2 = spmm(adj, s2, b1r)
    return norm(h2, cs2)


# speedup vs baseline: 1.1135x; 1.1046x over previous
import functools

import jax
import jax.numpy as jnp
from jax.experimental import pallas as pl
from jax.experimental.pallas import tpu as pltpu


def _gcn_body(adj_ref, x_ref, w0_ref, w1_ref, b0_ref, b1_ref, out_ref,
              s_ref, h_ref, cs_ref, *, n_rows, bm, n_blocks):
    t = pl.program_id(0)
    p = n_blocks
    inv_n = 1.0 / n_rows

    @pl.when(t == 0)
    def _start_layer0():
        s_ref[...] = jnp.dot(x_ref[...], w0_ref[...],
                             preferred_element_type=jnp.float32)
        cs_ref[...] = jnp.zeros_like(cs_ref)

    @pl.when(t == p)
    def _start_layer1():
        x = h_ref[...] - cs_ref[...] * inv_n
        rn = jnp.sqrt(1e-6 + jnp.sum(x * x, axis=1, keepdims=True))
        s_ref[...] = jnp.dot(x / rn, w1_ref[...],
                             preferred_element_type=jnp.float32)
        cs_ref[...] = jnp.zeros_like(cs_ref)

    @pl.when(t < 2 * p)
    def _mm_step():
        i = jnp.where(t < p, t, t - p)
        b = jnp.where(t < p, b0_ref[...], b1_ref[...])
        h = jnp.dot(adj_ref[...], s_ref[...],
                    preferred_element_type=jnp.float32)
        h = jnp.maximum(h + b, 0.0)
        h_ref[pl.ds(i * bm, bm), :] = h
        cs_ref[...] += jnp.sum(h, axis=0, keepdims=True)

    @pl.when(t == 2 * p)
    def _finish():
        x = h_ref[...] - cs_ref[...] * inv_n
        rn = jnp.sqrt(1e-6 + jnp.sum(x * x, axis=1, keepdims=True))
        out_ref[...] = x / rn


def _pick_block(n, target):
    best = 8
    for bm in range(8, min(n, target) + 1, 8):
        if n % bm == 0:
            best = bm
    return best


def kernel(in_feature, adj, W0, b0, W1, b1):
    n, d = in_feature.shape
    bm = _pick_block(n, 400)
    p = n // bm

    def adj_index(t):
        return (jnp.where(t < p, t, jnp.where(t < 2 * p, t - p, p - 1)), 0)

    full = lambda t: (0, 0)

    return pl.pallas_call(
        functools.partial(_gcn_body, n_rows=n, bm=bm, n_blocks=p),
        grid=(2 * p + 1,),
        in_specs=[
            pl.BlockSpec((bm, n), adj_index),
            pl.BlockSpec((n, d), full),
            pl.BlockSpec((d, d), full),
            pl.BlockSpec((d, d), full),
            pl.BlockSpec((1, d), full),
            pl.BlockSpec((1, d), full),
        ],
        out_specs=pl.BlockSpec((n, d), full),
        out_shape=jax.ShapeDtypeStruct((n, d), jnp.float32),
        scratch_shapes=[
            pltpu.VMEM((n, d), jnp.float32),
            pltpu.VMEM((n, d), jnp.float32),
            pltpu.VMEM((1, d), jnp.float32),
        ],
    )(adj, in_feature, W0, W1, b0.reshape(1, d), b1.reshape(1, d))


# speedup vs baseline: 1.1136x; 1.0001x over previous
import functools

import jax
import jax.numpy as jnp
from jax.experimental import pallas as pl
from jax.experimental.pallas import tpu as pltpu


def _gcn_body(adj_ref, x_ref, w0_ref, w1_ref, b0_ref, b1_ref, out_ref,
              s_ref, h_ref, cs_ref, *, n_rows, bm, n_blocks):
    t = pl.program_id(0)
    p = n_blocks
    inv_n = 1.0 / n_rows

    @pl.when(t == 0)
    def _start_layer0():
        s_ref[...] = jnp.dot(x_ref[...], w0_ref[...],
                             preferred_element_type=jnp.float32)
        cs_ref[...] = jnp.zeros_like(cs_ref)

    @pl.when(t == p)
    def _start_layer1():
        x = h_ref[...] - cs_ref[...] * inv_n
        rn = jnp.sqrt(1e-6 + jnp.sum(x * x, axis=1, keepdims=True))
        s_ref[...] = jnp.dot(x / rn, w1_ref[...],
                             preferred_element_type=jnp.float32)
        cs_ref[...] = jnp.zeros_like(cs_ref)

    @pl.when(t < 2 * p)
    def _mm_step():
        i = jnp.where(t < p, t, t - p)
        b = jnp.where(t < p, b0_ref[...], b1_ref[...])
        h = jnp.dot(adj_ref[...].astype(jnp.bfloat16),
                    s_ref[...].astype(jnp.bfloat16),
                    preferred_element_type=jnp.float32)
        h = jnp.maximum(h + b, 0.0)
        h_ref[pl.ds(i * bm, bm), :] = h
        cs_ref[...] += jnp.sum(h, axis=0, keepdims=True)

    @pl.when(t == 2 * p)
    def _finish():
        x = h_ref[...] - cs_ref[...] * inv_n
        rn = jnp.sqrt(1e-6 + jnp.sum(x * x, axis=1, keepdims=True))
        out_ref[...] = x / rn


def _pick_block(n, target):
    best = 8
    for bm in range(8, min(n, target) + 1, 8):
        if n % bm == 0:
            best = bm
    return best


def kernel(in_feature, adj, W0, b0, W1, b1):
    n, d = in_feature.shape
    bm = _pick_block(n, 400)
    p = n // bm

    def adj_index(t):
        return (jnp.where(t < p, t, jnp.where(t < 2 * p, t - p, p - 1)), 0)

    full = lambda t: (0, 0)

    return pl.pallas_call(
        functools.partial(_gcn_body, n_rows=n, bm=bm, n_blocks=p),
        grid=(2 * p + 1,),
        in_specs=[
            pl.BlockSpec((bm, n), adj_index),
            pl.BlockSpec((n, d), full),
            pl.BlockSpec((d, d), full),
            pl.BlockSpec((d, d), full),
            pl.BlockSpec((1, d), full),
            pl.BlockSpec((1, d), full),
        ],
        out_specs=pl.BlockSpec((n, d), full),
        out_shape=jax.ShapeDtypeStruct((n, d), jnp.float32),
        scratch_shapes=[
            pltpu.VMEM((n, d), jnp.float32),
            pltpu.VMEM((n, d), jnp.float32),
            pltpu.VMEM((1, d), jnp.float32),
        ],
    )(adj, in_feature, W0, W1, b0.reshape(1, d), b1.reshape(1, d))


# speedup vs baseline: 1.1139x; 1.0003x over previous
import functools

import jax
import jax.numpy as jnp
from jax.experimental import pallas as pl
from jax.experimental.pallas import tpu as pltpu


def _gcn_body(adjT_ref, adjB_ref, x_ref, w0_ref, w1_ref, b0_ref, b1_ref,
              out_ref, s_ref, h_ref, cs_ref, *, n_rows, bm, n_blocks):
    t = pl.program_id(0)
    p = n_blocks
    inv_n = 1.0 / n_rows

    @pl.when(t == 0)
    def _start_layer0():
        s_ref[...] = jnp.dot(x_ref[...], w0_ref[...],
                             preferred_element_type=jnp.float32)
        cs_ref[...] = jnp.zeros_like(cs_ref)

    @pl.when(t == p)
    def _start_layer1():
        x = h_ref[...] - cs_ref[...] * inv_n
        rn = jnp.sqrt(1e-6 + jnp.sum(x * x, axis=1, keepdims=True))
        s_ref[...] = jnp.dot(x / rn, w1_ref[...],
                             preferred_element_type=jnp.float32)
        cs_ref[...] = jnp.zeros_like(cs_ref)

    @pl.when(t < 2 * p)
    def _mm_step():
        i = jnp.where(t < p, t, t - p)
        b = jnp.where(t < p, b0_ref[...], b1_ref[...])
        hb = bm // 2
        ht = jnp.maximum(jnp.dot(adjT_ref[...], s_ref[...],
                                 preferred_element_type=jnp.float32) + b, 0.0)
        hbot = jnp.maximum(jnp.dot(adjB_ref[...], s_ref[...],
                                   preferred_element_type=jnp.float32) + b, 0.0)
        h_ref[pl.ds(i * bm, hb), :] = ht
        h_ref[pl.ds(i * bm + hb, hb), :] = hbot
        cs_ref[...] += (jnp.sum(ht, axis=0, keepdims=True) +
                        jnp.sum(hbot, axis=0, keepdims=True))

    @pl.when(t == 2 * p)
    def _finish():
        x = h_ref[...] - cs_ref[...] * inv_n
        rn = jnp.sqrt(1e-6 + jnp.sum(x * x, axis=1, keepdims=True))
        out_ref[...] = x / rn


def _pick_block(n, target):
    best = 8
    for bm in range(8, min(n, target) + 1, 8):
        if n % bm == 0:
            best = bm
    return best


def kernel(in_feature, adj, W0, b0, W1, b1):
    n, d = in_feature.shape
    bm = _pick_block(n, 400)
    p = n // bm

    def adj_index(t):
        return (jnp.where(t < p, t, jnp.where(t < 2 * p, t - p, p - 1)), 0)

    full = lambda t: (0, 0)

    return pl.pallas_call(
        functools.partial(_gcn_body, n_rows=n, bm=bm, n_blocks=p),
        grid=(2 * p + 1,),
        in_specs=[
            pl.BlockSpec((bm // 2, n), lambda t: (2 * adj_index(t)[0], 0)),
            pl.BlockSpec((bm // 2, n), lambda t: (2 * adj_index(t)[0] + 1, 0)),
            pl.BlockSpec((n, d), full),
            pl.BlockSpec((d, d), full),
            pl.BlockSpec((d, d), full),
            pl.BlockSpec((1, d), full),
            pl.BlockSpec((1, d), full),
        ],
        out_specs=pl.BlockSpec((n, d), full),
        out_shape=jax.ShapeDtypeStruct((n, d), jnp.float32),
        scratch_shapes=[
            pltpu.VMEM((n, d), jnp.float32),
            pltpu.VMEM((n, d), jnp.float32),
            pltpu.VMEM((1, d), jnp.float32),
        ],
    )(adj, adj, in_feature, W0, W1, b0.reshape(1, d), b1.reshape(1, d))
